# baseline (speedup 1.0000x reference)
import functools

import jax
import jax.numpy as jnp
from jax import lax
from jax.experimental import pallas as pl
from jax.experimental.pallas import tpu as pltpu
from jax.experimental.pallas import tpu_sc as plsc


def _mm_body(x_ref, w_ref, o_ref):
    o_ref[...] = jnp.dot(x_ref[...], w_ref[...], preferred_element_type=jnp.float32)


def _mm(x, w):
    n, d = x.shape
    return pl.pallas_call(
        _mm_body,
        out_shape=jax.ShapeDtypeStruct((n, w.shape[1]), jnp.float32),
    )(x, w)


def _mm_bias_body(x_ref, w_ref, b_ref, o_ref):
    o_ref[...] = (
        jnp.dot(x_ref[...], w_ref[...], preferred_element_type=jnp.float32)
        + b_ref[...]
    )


def _mm_bias_pad(x, w, b, block, out_rows):
    n, d = x.shape
    h = w.shape[1]
    grid = out_rows // block
    nblk = n // block

    def in_map(i):
        return (jnp.minimum(i, nblk - 1), 0)

    return pl.pallas_call(
        _mm_bias_body,
        grid=(grid,),
        in_specs=[
            pl.BlockSpec((block, d), in_map),
            pl.BlockSpec((d, h), lambda i: (0, 0)),
            pl.BlockSpec((1, h), lambda i: (0, 0)),
        ],
        out_specs=pl.BlockSpec((block, h), lambda i: (i, 0)),
        out_shape=jax.ShapeDtypeStruct((out_rows, h), jnp.float32),
    )(x, w, b.reshape(1, -1))


def _mlp_body(au_ref, af_ref, w1u_ref, b1u_ref, w2u_ref, b2u_ref,
              w1f_ref, b1f_ref, w2f_ref, b2f_ref, wc1_ref, wc2_ref, bc_ref,
              o_ref):
    f32 = jnp.float32
    hu = jnp.maximum(
        jnp.dot(au_ref[...], w1u_ref[...], preferred_element_type=f32)
        + b1u_ref[...], 0.0)
    ou = jnp.maximum(
        jnp.dot(hu, w2u_ref[...], preferred_element_type=f32) + b2u_ref[...],
        0.0)
    hf = jnp.maximum(
        jnp.dot(af_ref[...], w1f_ref[...], preferred_element_type=f32)
        + b1f_ref[...], 0.0)
    of = jnp.maximum(
        jnp.dot(hf, w2f_ref[...], preferred_element_type=f32) + b2f_ref[...],
        0.0)
    o_ref[...] = jnp.maximum(
        jnp.dot(ou, wc1_ref[...], preferred_element_type=f32)
        + jnp.dot(of, wc2_ref[...], preferred_element_type=f32)
        + bc_ref[...], 0.0)


def _mlps(acc_up, acc_f, W1u, b1u, W2u, b2u, W1f, b1f, W2f, b2f, Wc, bc,
          block):
    n, d = acc_up.shape
    h = W1u.shape[1]
    grid = n // block
    wspec = pl.BlockSpec((d, h), lambda i: (0, 0))
    bspec = pl.BlockSpec((1, h), lambda i: (0, 0))
    nspec = pl.BlockSpec((block, d), lambda i: (i, 0))
    return pl.pallas_call(
        _mlp_body,
        grid=(grid,),
        in_specs=[nspec, nspec,
                  wspec, bspec, wspec, bspec,
                  wspec, bspec, wspec, bspec,
                  wspec, wspec, bspec],
        out_specs=pl.BlockSpec((block, h), lambda i: (i, 0)),
        out_shape=jax.ShapeDtypeStruct((n, h), jnp.float32),
    )(acc_up, acc_f,
      W1u, b1u.reshape(1, -1), W2u, b2u.reshape(1, -1),
      W1f, b1f.reshape(1, -1), W2f, b2f.reshape(1, -1),
      Wc[:d], Wc[d:], bc.reshape(1, -1))


def _sc_task(table, aw, x, src3, dst3):
    n, d = x.shape
    ns_chk, nch, ch = src3.shape
    e = ns_chk * nch * ch
    info = plsc.get_sparse_core_info()
    ns = info.num_subcores
    ept = e // ns
    half = n // 2
    trash = half
    rpt = (half // ns) // 8 * 8
    tail = half - ns * rpt

    mesh = plsc.VectorSubcoreMesh(core_axis_name="c", subcore_axis_name="s")
    have_aw = aw is not None

    buf_shape = pltpu.VMEM((ch, d), jnp.float32)
    scratch = [
        pltpu.VMEM((nch, ch), jnp.int32),
        pltpu.VMEM((nch, ch), jnp.int32),
        buf_shape, buf_shape,
    ] + ([buf_shape] if have_aw else []) + [
        pltpu.VMEM_SHARED((half + 8, d), jnp.float32),
    ] + [pltpu.SemaphoreType.DMA] * (3 if have_aw else 2)

    @functools.partial(
        pl.kernel,
        mesh=mesh,
        out_type=jax.ShapeDtypeStruct((n, d), jnp.float32),
        scratch_types=scratch,
    )
    def k(table_hbm, aw_hbm, src_hbm, dst_hbm, x_hbm, out, srcb, dstb, *rest):
        if have_aw:
            g0, g1, ab, acc, gs0, gs1, asem = rest
        else:
            g0, g1, acc, gs0, gs1 = rest
        gbuf, gsem = (g0, g1), (gs0, gs1)
        cid = lax.axis_index("c")
        sid = lax.axis_index("s")
        lo = cid * half
        r0 = sid * rpt

        pltpu.sync_copy(x_hbm.at[pl.ds(lo + r0, rpt)], acc.at[pl.ds(r0, rpt)])

        @pl.when(sid == ns - 1)
        def _init_tail():
            t = ns * rpt
            pltpu.sync_copy(x_hbm.at[pl.ds(lo + t, tail)],
                            acc.at[pl.ds(t, tail)])

        pltpu.sync_copy(src_hbm.at[sid], srcb)
        pltpu.sync_copy(dst_hbm.at[sid], dstb)

        @plsc.parallel_loop(0, nch, 1, unroll=2)
        def _clamp(c):
            for kk in range(ch // 16):
                s = pl.ds(kk * 16, 16)
                dv = dstb[c, s]
                keep = (dv >= lo) & (dv < lo + half)
                dstb[c, s] = jnp.where(keep, dv - lo, trash)

        plsc.subcore_barrier()

        ebase = sid * ept

        def gcopy(c, b):
            return pltpu.make_async_copy(
                table_hbm.at[srcb.at[c]], gbuf[b], gsem[b])

        def acopy(c):
            return pltpu.make_async_copy(
                aw_hbm.at[pl.ds(ebase + c * ch, ch)], ab, asem)

        gcopy(0, 0).start()
        gcopy(1, 1).start()
        if have_aw:
            acopy(0).start()

        @pl.loop(0, nch, step=2)
        def _(c):
            for b in range(2):
                cc = c + b
                gcopy(cc, b).wait()
                if have_aw:
                    acopy(cc).wait()

                    @plsc.parallel_loop(0, ch, 1, unroll=2)
                    def _(r):
                        for kk in range(d // 16):
                            s = pl.ds(kk * 16, 16)
                            gbuf[b][r, s] = jnp.maximum(
                                gbuf[b][r, s] + ab[r, s], 0.0)

                    nxt1 = cc + 1

                    @pl.when(nxt1 < nch)
                    def _():
                        acopy(nxt1).start()

                pltpu.sync_copy(gbuf[b], acc.at[dstb.at[cc]], add=True)
                nxt2 = cc + 2

                @pl.when(nxt2 < nch)
                def _():
                    gcopy(nxt2, b).start()

        plsc.subcore_barrier()

        pltpu.sync_copy(acc.at[pl.ds(r0, rpt)], out.at[pl.ds(lo + r0, rpt)])

        @pl.when(sid == ns - 1)
        def _flush_tail():
            t = ns * rpt
            pltpu.sync_copy(acc.at[pl.ds(t, tail)],
                            out.at[pl.ds(lo + t, tail)])

    return k(table, aw if have_aw else table, src3, dst3, x)


_CH = 128


def _pad_idx(idx, n, ns, nch, ch):
    e = idx.shape[1]
    pad = ns * nch * ch - e
    src = jnp.concatenate([idx[0], jnp.zeros((pad,), jnp.int32)])
    dst = jnp.concatenate([idx[1], jnp.full((pad,), n, jnp.int32)])
    return src.reshape(ns, nch, ch), dst.reshape(ns, nch, ch)


def kernel(x, up_index, up_attr, face_index, face_attr,
           W_msg, b_msg, W1u, b1u, W2u, b2u, W1f, b1f, W2f, b2f, Wc, bc):
    n, d = x.shape
    e = up_attr.shape[0]

    ns = plsc.get_sparse_core_info().num_subcores
    nch = -(-e // (ns * _CH))
    nch += nch % 2
    usrc3, udst3 = _pad_idx(up_index, n, ns, nch, _CH)
    fsrc3, fdst3 = _pad_idx(face_index, n, ns, nch, _CH)

    acc_f = _sc_task(face_attr, None, x, fsrc3, fdst3)

    xw = _mm(x, W_msg[:d])
    blk = 2000
    aw_rows = -(-(ns * nch * _CH) // blk) * blk
    aw = _mm_bias_pad(up_attr, W_msg[d:], b_msg, blk, aw_rows)

    acc_up = _sc_task(xw, aw, x, usrc3, udst3)

    return _mlps(acc_up, acc_f, W1u, b1u, W2u, b2u, W1f, b1f, W2f, b2f,
                 Wc, bc, block=1000)


# speedup vs baseline: 1.0646x; 1.0646x over previous
import functools

import jax
import jax.numpy as jnp
from jax import lax
from jax.experimental import pallas as pl
from jax.experimental.pallas import tpu as pltpu
from jax.experimental.pallas import tpu_sc as plsc


def _mm_body(x_ref, w_ref, o_ref):
    o_ref[...] = jnp.dot(x_ref[...], w_ref[...], preferred_element_type=jnp.float32)


def _mm(x, w):
    n, d = x.shape
    return pl.pallas_call(
        _mm_body,
        out_shape=jax.ShapeDtypeStruct((n, w.shape[1]), jnp.float32),
    )(x, w)


def _mm_bias_body(x_ref, w_ref, b_ref, o_ref):
    o_ref[...] = (
        jnp.dot(x_ref[...], w_ref[...], preferred_element_type=jnp.float32)
        + b_ref[...]
    )


def _mm_bias_pad(x, w, b, block, out_rows):
    n, d = x.shape
    h = w.shape[1]
    grid = out_rows // block
    nblk = n // block

    def in_map(i):
        return (jnp.minimum(i, nblk - 1), 0)

    return pl.pallas_call(
        _mm_bias_body,
        grid=(grid,),
        in_specs=[
            pl.BlockSpec((block, d), in_map),
            pl.BlockSpec((d, h), lambda i: (0, 0)),
            pl.BlockSpec((1, h), lambda i: (0, 0)),
        ],
        out_specs=pl.BlockSpec((block, h), lambda i: (i, 0)),
        out_shape=jax.ShapeDtypeStruct((out_rows, h), jnp.float32),
    )(x, w, b.reshape(1, -1))


def _mlp_body(au_ref, af0_ref, af1_ref, x_ref,
              w1u_ref, b1u_ref, w2u_ref, b2u_ref,
              w1f_ref, b1f_ref, w2f_ref, b2f_ref, wc1_ref, wc2_ref, bc_ref,
              o_ref):
    f32 = jnp.float32
    hu = jnp.maximum(
        jnp.dot(au_ref[...], w1u_ref[...], preferred_element_type=f32)
        + b1u_ref[...], 0.0)
    ou = jnp.maximum(
        jnp.dot(hu, w2u_ref[...], preferred_element_type=f32) + b2u_ref[...],
        0.0)
    af = af0_ref[0] + af1_ref[0] - x_ref[...]
    hf = jnp.maximum(
        jnp.dot(af, w1f_ref[...], preferred_element_type=f32)
        + b1f_ref[...], 0.0)
    of = jnp.maximum(
        jnp.dot(hf, w2f_ref[...], preferred_element_type=f32) + b2f_ref[...],
        0.0)
    o_ref[...] = jnp.maximum(
        jnp.dot(ou, wc1_ref[...], preferred_element_type=f32)
        + jnp.dot(of, wc2_ref[...], preferred_element_type=f32)
        + bc_ref[...], 0.0)


def _mlps(acc_up, acc_f2, x, W1u, b1u, W2u, b2u, W1f, b1f, W2f, b2f, Wc, bc,
          block):
    n, d = acc_up.shape
    h = W1u.shape[1]
    grid = n // block
    wspec = pl.BlockSpec((d, h), lambda i: (0, 0))
    bspec = pl.BlockSpec((1, h), lambda i: (0, 0))
    nspec = pl.BlockSpec((block, d), lambda i: (i, 0))
    f0spec = pl.BlockSpec((1, block, d), lambda i: (0, i, 0))
    f1spec = pl.BlockSpec((1, block, d), lambda i: (1, i, 0))
    return pl.pallas_call(
        _mlp_body,
        grid=(grid,),
        in_specs=[nspec, f0spec, f1spec, nspec,
                  wspec, bspec, wspec, bspec,
                  wspec, bspec, wspec, bspec,
                  wspec, wspec, bspec],
        out_specs=pl.BlockSpec((block, h), lambda i: (i, 0)),
        out_shape=jax.ShapeDtypeStruct((n, h), jnp.float32),
    )(acc_up, acc_f2, acc_f2, x,
      W1u, b1u.reshape(1, -1), W2u, b2u.reshape(1, -1),
      W1f, b1f.reshape(1, -1), W2f, b2f.reshape(1, -1),
      Wc[:d], Wc[d:], bc.reshape(1, -1))


def _sc_task(table, aw, x, src3, dst3):
    n, d = x.shape
    ns_chk, nch, ch = src3.shape
    e = ns_chk * nch * ch
    info = plsc.get_sparse_core_info()
    ns = info.num_subcores
    ept = e // ns
    half = n // 2
    trash = half
    rpt = (half // ns) // 8 * 8
    tail = half - ns * rpt

    mesh = plsc.VectorSubcoreMesh(core_axis_name="c", subcore_axis_name="s")
    have_aw = aw is not None

    buf_shape = pltpu.VMEM((ch, d), jnp.float32)
    scratch = [
        pltpu.VMEM((nch, ch), jnp.int32),
        pltpu.VMEM((nch, ch), jnp.int32),
        buf_shape, buf_shape,
    ] + ([buf_shape] if have_aw else []) + [
        pltpu.VMEM_SHARED((half + 8, d), jnp.float32),
    ] + [pltpu.SemaphoreType.DMA] * (3 if have_aw else 2)

    @functools.partial(
        pl.kernel,
        mesh=mesh,
        out_type=jax.ShapeDtypeStruct((n, d), jnp.float32),
        scratch_types=scratch,
    )
    def k(table_hbm, aw_hbm, src_hbm, dst_hbm, x_hbm, out, srcb, dstb, *rest):
        if have_aw:
            g0, g1, ab, acc, gs0, gs1, asem = rest
        else:
            g0, g1, acc, gs0, gs1 = rest
        gbuf, gsem = (g0, g1), (gs0, gs1)
        cid = lax.axis_index("c")
        sid = lax.axis_index("s")
        lo = cid * half
        r0 = sid * rpt

        pltpu.sync_copy(x_hbm.at[pl.ds(lo + r0, rpt)], acc.at[pl.ds(r0, rpt)])

        @pl.when(sid == ns - 1)
        def _init_tail():
            t = ns * rpt
            pltpu.sync_copy(x_hbm.at[pl.ds(lo + t, tail)],
                            acc.at[pl.ds(t, tail)])

        pltpu.sync_copy(src_hbm.at[sid], srcb)
        pltpu.sync_copy(dst_hbm.at[sid], dstb)

        @plsc.parallel_loop(0, nch, 1, unroll=2)
        def _clamp(c):
            for kk in range(ch // 16):
                s = pl.ds(kk * 16, 16)
                dv = dstb[c, s]
                keep = (dv >= lo) & (dv < lo + half)
                dstb[c, s] = jnp.where(keep, dv - lo, trash)

        plsc.subcore_barrier()

        ebase = sid * ept

        def gcopy(c, b):
            return pltpu.make_async_copy(
                table_hbm.at[srcb.at[c]], gbuf[b], gsem[b])

        def acopy(c):
            return pltpu.make_async_copy(
                aw_hbm.at[pl.ds(ebase + c * ch, ch)], ab, asem)

        gcopy(0, 0).start()
        gcopy(1, 1).start()
        if have_aw:
            acopy(0).start()

        @pl.loop(0, nch, step=2)
        def _(c):
            for b in range(2):
                cc = c + b
                gcopy(cc, b).wait()
                if have_aw:
                    acopy(cc).wait()

                    @plsc.parallel_loop(0, ch, 1, unroll=2)
                    def _(r):
                        for kk in range(d // 16):
                            s = pl.ds(kk * 16, 16)
                            gbuf[b][r, s] = jnp.maximum(
                                gbuf[b][r, s] + ab[r, s], 0.0)

                    nxt1 = cc + 1

                    @pl.when(nxt1 < nch)
                    def _():
                        acopy(nxt1).start()

                pltpu.sync_copy(gbuf[b], acc.at[dstb.at[cc]], add=True)
                nxt2 = cc + 2

                @pl.when(nxt2 < nch)
                def _():
                    gcopy(nxt2, b).start()

        plsc.subcore_barrier()

        pltpu.sync_copy(acc.at[pl.ds(r0, rpt)], out.at[pl.ds(lo + r0, rpt)])

        @pl.when(sid == ns - 1)
        def _flush_tail():
            t = ns * rpt
            pltpu.sync_copy(acc.at[pl.ds(t, tail)],
                            out.at[pl.ds(lo + t, tail)])

    return k(table, aw if have_aw else table, src3, dst3, x)


def _sc_face(table, x, src4, dst4):
    n, d = x.shape
    nc, ns_chk, nchr, chw = src4.shape
    half = chw // 2
    info = plsc.get_sparse_core_info()
    ns = info.num_subcores
    rpt = (n // ns) // 8 * 8
    tail = n - ns * rpt

    mesh = plsc.VectorSubcoreMesh(core_axis_name="c", subcore_axis_name="s")

    scratch = [
        pltpu.VMEM((nchr, chw), jnp.int32),
        pltpu.VMEM((2 * nchr, half), jnp.int32),
        pltpu.VMEM((half, d), jnp.float32),
        pltpu.VMEM((half, d), jnp.float32),
        pltpu.VMEM_SHARED((n + 8, d), jnp.float32),
    ] + [pltpu.SemaphoreType.DMA] * 2

    @functools.partial(
        pl.kernel,
        mesh=mesh,
        out_type=jax.ShapeDtypeStruct((2, n, d), jnp.float32),
        scratch_types=scratch,
    )
    def k(table_hbm, src_hbm, dst_hbm, x_hbm, out, srcb, dstb, g0, g1,
          acc, gs0, gs1):
        gbuf, gsem = (g0, g1), (gs0, gs1)
        cid = lax.axis_index("c")
        sid = lax.axis_index("s")
        r0 = sid * rpt

        pltpu.sync_copy(x_hbm.at[pl.ds(r0, rpt)], acc.at[pl.ds(r0, rpt)])

        @pl.when(sid == ns - 1)
        def _init_tail():
            t = ns * rpt
            pltpu.sync_copy(x_hbm.at[pl.ds(t, tail)], acc.at[pl.ds(t, tail)])

        pltpu.sync_copy(src_hbm.at[cid, sid], srcb)
        pltpu.sync_copy(dst_hbm.at[cid, sid], dstb)

        plsc.subcore_barrier()

        def gcopy(p, b):
            return pltpu.make_async_copy(
                table_hbm.at[srcb.at[p, pl.ds(b * half, half)]],
                gbuf[b], gsem[b])

        gcopy(0, 0).start()
        gcopy(0, 1).start()

        @pl.loop(0, nchr)
        def _(p):
            for b in range(2):
                gcopy(p, b).wait()
                pltpu.sync_copy(gbuf[b], acc.at[dstb.at[2 * p + b]],
                                add=True)

                @pl.when(p + 1 < nchr)
                def _():
                    gcopy(p + 1, b).start()

        plsc.subcore_barrier()

        pltpu.sync_copy(acc.at[pl.ds(r0, rpt)],
                        out.at[cid, pl.ds(r0, rpt)])

        @pl.when(sid == ns - 1)
        def _flush_tail():
            t = ns * rpt
            pltpu.sync_copy(acc.at[pl.ds(t, tail)],
                            out.at[cid, pl.ds(t, tail)])

    return k(table, src4, dst4, x)


_CH = 128


def _pad_idx(idx, n, ns, nch, ch):
    e = idx.shape[1]
    pad = ns * nch * ch - e
    src = jnp.concatenate([idx[0], jnp.zeros((pad,), jnp.int32)])
    dst = jnp.concatenate([idx[1], jnp.full((pad,), n, jnp.int32)])
    return src.reshape(ns, nch, ch), dst.reshape(ns, nch, ch)


def _split_idx(idx, n, ns, nchr):
    e = idx.shape[1]
    ept = e // 2 // ns
    pad = nchr * 128 - ept
    src = jnp.pad(idx[0].reshape(2, ns, ept), ((0, 0), (0, 0), (0, pad)))
    dst = jnp.pad(idx[1].reshape(2, ns, ept), ((0, 0), (0, 0), (0, pad)),
                  constant_values=n)
    return (src.reshape(2, ns, nchr, 128),
            dst.reshape(2, ns, 2 * nchr, 64))


def kernel(x, up_index, up_attr, face_index, face_attr,
           W_msg, b_msg, W1u, b1u, W2u, b2u, W1f, b1f, W2f, b2f, Wc, bc):
    n, d = x.shape
    e = up_attr.shape[0]

    ns = plsc.get_sparse_core_info().num_subcores
    nch = -(-e // (ns * _CH))
    nch += nch % 2
    usrc3, udst3 = _pad_idx(up_index, n, ns, nch, _CH)
    fnchr = -(-(e // 2 // ns) // _CH)
    fnchr += fnchr % 2
    fsrc4, fdst4 = _split_idx(face_index, n, ns, fnchr)

    acc_f2 = _sc_face(face_attr, x, fsrc4, fdst4)

    xw = _mm(x, W_msg[:d])
    blk = 2000
    aw_rows = -(-(ns * nch * _CH) // blk) * blk
    aw = _mm_bias_pad(up_attr, W_msg[d:], b_msg, blk, aw_rows)

    acc_up = _sc_task(xw, aw, x, usrc3, udst3)

    return _mlps(acc_up, acc_f2, x, W1u, b1u, W2u, b2u, W1f, b1f, W2f, b2f,
                 Wc, bc, block=1000)


# speedup vs baseline: 1.0757x; 1.0105x over previous
import functools

import jax
import jax.numpy as jnp
from jax import lax
from jax.experimental import pallas as pl
from jax.experimental.pallas import tpu as pltpu
from jax.experimental.pallas import tpu_sc as plsc


def _mm_body(x_ref, w_ref, o_ref):
    o_ref[...] = jnp.dot(x_ref[...], w_ref[...], preferred_element_type=jnp.float32)


def _mm(x, w):
    n, d = x.shape
    return pl.pallas_call(
        _mm_body,
        out_shape=jax.ShapeDtypeStruct((n, w.shape[1]), jnp.float32),
    )(x, w)


def _mm_bias_body(x_ref, w_ref, b_ref, o_ref):
    o_ref[...] = (
        jnp.dot(x_ref[...], w_ref[...], preferred_element_type=jnp.float32)
        + b_ref[...]
    )


def _mm_bias(x, w, b, block):
    n, d = x.shape
    h = w.shape[1]
    grid = n // block
    return pl.pallas_call(
        _mm_bias_body,
        grid=(grid,),
        in_specs=[
            pl.BlockSpec((block, d), lambda i: (i, 0)),
            pl.BlockSpec((d, h), lambda i: (0, 0)),
            pl.BlockSpec((1, h), lambda i: (0, 0)),
        ],
        out_specs=pl.BlockSpec((block, h), lambda i: (i, 0)),
        out_shape=jax.ShapeDtypeStruct((n, h), jnp.float32),
    )(x, w, b.reshape(1, -1))


def _mlp_body(au_ref, af_ref, w1u_ref, b1u_ref, w2u_ref, b2u_ref,
              w1f_ref, b1f_ref, w2f_ref, b2f_ref, wc1_ref, wc2_ref, bc_ref,
              o_ref):
    f32 = jnp.float32
    hu = jnp.maximum(
        jnp.dot(au_ref[...], w1u_ref[...], preferred_element_type=f32)
        + b1u_ref[...], 0.0)
    ou = jnp.maximum(
        jnp.dot(hu, w2u_ref[...], preferred_element_type=f32) + b2u_ref[...],
        0.0)
    hf = jnp.maximum(
        jnp.dot(af_ref[...], w1f_ref[...], preferred_element_type=f32)
        + b1f_ref[...], 0.0)
    of = jnp.maximum(
        jnp.dot(hf, w2f_ref[...], preferred_element_type=f32) + b2f_ref[...],
        0.0)
    o_ref[...] = jnp.maximum(
        jnp.dot(ou, wc1_ref[...], preferred_element_type=f32)
        + jnp.dot(of, wc2_ref[...], preferred_element_type=f32)
        + bc_ref[...], 0.0)


def _mlps(acc_up, acc_f, W1u, b1u, W2u, b2u, W1f, b1f, W2f, b2f, Wc, bc,
          block):
    n, d = acc_up.shape
    h = W1u.shape[1]
    grid = n // block
    wspec = pl.BlockSpec((d, h), lambda i: (0, 0))
    bspec = pl.BlockSpec((1, h), lambda i: (0, 0))
    nspec = pl.BlockSpec((block, d), lambda i: (i, 0))
    return pl.pallas_call(
        _mlp_body,
        grid=(grid,),
        in_specs=[nspec, nspec,
                  wspec, bspec, wspec, bspec,
                  wspec, bspec, wspec, bspec,
                  wspec, wspec, bspec],
        out_specs=pl.BlockSpec((block, h), lambda i: (i, 0)),
        out_shape=jax.ShapeDtypeStruct((n, h), jnp.float32),
    )(acc_up, acc_f,
      W1u, b1u.reshape(1, -1), W2u, b2u.reshape(1, -1),
      W1f, b1f.reshape(1, -1), W2f, b2f.reshape(1, -1),
      Wc[:d], Wc[d:], bc.reshape(1, -1))


_CH = 80


def _sc_both(face_attr, xw, aw, x, fsrc3, fdst3, usrc3, udst3):
    n, d = x.shape
    ns_chk, nch, ch = usrc3.shape
    e = ns_chk * nch * ch
    info = plsc.get_sparse_core_info()
    ns = info.num_subcores
    ept = e // ns
    half = n // 2
    trash = half
    rpt = (half // ns) // 8 * 8
    tail = half - ns * rpt

    mesh = plsc.VectorSubcoreMesh(core_axis_name="c", subcore_axis_name="s")

    buf_shape = pltpu.VMEM((ch, d), jnp.float32)
    scratch = [
        pltpu.VMEM((nch, ch), jnp.int32),
        pltpu.VMEM((nch, ch), jnp.int32),
        buf_shape,
        buf_shape,
        pltpu.VMEM_SHARED((half + 8, d), jnp.float32),
    ] + [pltpu.SemaphoreType.DMA] * 2

    out_sds = jax.ShapeDtypeStruct((n, d), jnp.float32)

    @functools.partial(
        pl.kernel,
        mesh=mesh,
        out_type=(out_sds, out_sds),
        scratch_types=scratch,
    )
    def k(fa_hbm, xw_hbm, aw_hbm, fsrc_hbm, fdst_hbm, usrc_hbm, udst_hbm,
          x_hbm, out_f, out_up, srcb, dstb, gbuf, abuf, acc, gsem, asem):
        cid = lax.axis_index("c")
        sid = lax.axis_index("s")
        lo = cid * half
        r0 = sid * rpt
        ebase = sid * ept

        for table_hbm, src_hbm, dst_hbm, out, haw in (
                (fa_hbm, fsrc_hbm, fdst_hbm, out_f, False),
                (xw_hbm, usrc_hbm, udst_hbm, out_up, True)):

            pltpu.sync_copy(x_hbm.at[pl.ds(lo + r0, rpt)],
                            acc.at[pl.ds(r0, rpt)])

            @pl.when(sid == ns - 1)
            def _init_tail():
                t = ns * rpt
                pltpu.sync_copy(x_hbm.at[pl.ds(lo + t, tail)],
                                acc.at[pl.ds(t, tail)])

            pltpu.sync_copy(src_hbm.at[sid], srcb)
            pltpu.sync_copy(dst_hbm.at[sid], dstb)

            @plsc.parallel_loop(0, nch, 1, unroll=2)
            def _clamp(c):
                for kk in range(ch // 16):
                    s = pl.ds(kk * 16, 16)
                    dv = dstb[c, s]
                    keep = (dv >= lo) & (dv < lo + half)
                    dstb[c, s] = jnp.where(keep, dv - lo, trash)

            plsc.subcore_barrier()

            @pl.loop(0, nch)
            def _(c):
                gd = pltpu.async_copy(table_hbm.at[srcb.at[c]], gbuf, gsem)
                if haw:
                    ad = pltpu.async_copy(
                        aw_hbm.at[pl.ds(ebase + c * ch, ch)], abuf, asem)
                gd.wait()
                if haw:
                    ad.wait()

                    @plsc.parallel_loop(0, ch, 1, unroll=2)
                    def _(r):
                        for kk in range(d // 16):
                            s = pl.ds(kk * 16, 16)
                            abuf[r, s] = jnp.maximum(
                                abuf[r, s] + gbuf[r, s], 0.0)

                    pltpu.sync_copy(abuf, acc.at[dstb.at[c]], add=True)
                else:
                    pltpu.sync_copy(gbuf, acc.at[dstb.at[c]], add=True)

            plsc.subcore_barrier()

            pltpu.sync_copy(acc.at[pl.ds(r0, rpt)],
                            out.at[pl.ds(lo + r0, rpt)])

            @pl.when(sid == ns - 1)
            def _flush_tail():
                t = ns * rpt
                pltpu.sync_copy(acc.at[pl.ds(t, tail)],
                                out.at[pl.ds(lo + t, tail)])

            plsc.subcore_barrier()

    return k(face_attr, xw, aw, fsrc3, fdst3, usrc3, udst3, x)


def _idx3(idx, ns, nch, ch):
    return idx[0].reshape(ns, nch, ch), idx[1].reshape(ns, nch, ch)


def kernel(x, up_index, up_attr, face_index, face_attr,
           W_msg, b_msg, W1u, b1u, W2u, b2u, W1f, b1f, W2f, b2f, Wc, bc):
    n, d = x.shape
    e = up_attr.shape[0]

    ns = plsc.get_sparse_core_info().num_subcores
    nch = e // (ns * _CH)
    usrc3, udst3 = _idx3(up_index, ns, nch, _CH)
    fsrc3, fdst3 = _idx3(face_index, ns, nch, _CH)

    xw = _mm(x, W_msg[:d])
    aw = _mm_bias(up_attr, W_msg[d:], b_msg, block=2000)

    acc_f, acc_up = _sc_both(face_attr, xw, aw, x,
                             fsrc3, fdst3, usrc3, udst3)

    return _mlps(acc_up, acc_f, W1u, b1u, W2u, b2u, W1f, b1f, W2f, b2f,
                 Wc, bc, block=1000)


# speedup vs baseline: 1.0791x; 1.0032x over previous
import functools

import jax
import jax.numpy as jnp
from jax import lax
from jax.experimental import pallas as pl
from jax.experimental.pallas import tpu as pltpu
from jax.experimental.pallas import tpu_sc as plsc


def _mm_body(x_ref, w_ref, o_ref):
    o_ref[...] = jnp.dot(x_ref[...], w_ref[...], preferred_element_type=jnp.float32)


def _mm(x, w):
    n, d = x.shape
    return pl.pallas_call(
        _mm_body,
        out_shape=jax.ShapeDtypeStruct((n, w.shape[1]), jnp.float32),
    )(x, w)


def _mm_bias_body(x_ref, w_ref, b_ref, o_ref):
    o_ref[...] = (
        jnp.dot(x_ref[...], w_ref[...], preferred_element_type=jnp.float32)
        + b_ref[...]
    )


def _mm_bias(x, w, b, block):
    n, d = x.shape
    h = w.shape[1]
    grid = n // block
    return pl.pallas_call(
        _mm_bias_body,
        grid=(grid,),
        in_specs=[
            pl.BlockSpec((block, d), lambda i: (i, 0)),
            pl.BlockSpec((d, h), lambda i: (0, 0)),
            pl.BlockSpec((1, h), lambda i: (0, 0)),
        ],
        out_specs=pl.BlockSpec((block, h), lambda i: (i, 0)),
        out_shape=jax.ShapeDtypeStruct((n, h), jnp.float32),
    )(x, w, b.reshape(1, -1))


def _mlp_body(au_ref, af_ref, w1u_ref, b1u_ref, w2u_ref, b2u_ref,
              w1f_ref, b1f_ref, w2f_ref, b2f_ref, wc1_ref, wc2_ref, bc_ref,
              o_ref):
    f32 = jnp.float32
    hu = jnp.maximum(
        jnp.dot(au_ref[...], w1u_ref[...], preferred_element_type=f32)
        + b1u_ref[...], 0.0)
    ou = jnp.maximum(
        jnp.dot(hu, w2u_ref[...], preferred_element_type=f32) + b2u_ref[...],
        0.0)
    hf = jnp.maximum(
        jnp.dot(af_ref[...], w1f_ref[...], preferred_element_type=f32)
        + b1f_ref[...], 0.0)
    of = jnp.maximum(
        jnp.dot(hf, w2f_ref[...], preferred_element_type=f32) + b2f_ref[...],
        0.0)
    o_ref[...] = jnp.maximum(
        jnp.dot(ou, wc1_ref[...], preferred_element_type=f32)
        + jnp.dot(of, wc2_ref[...], preferred_element_type=f32)
        + bc_ref[...], 0.0)


def _mlps(acc_up, acc_f, W1u, b1u, W2u, b2u, W1f, b1f, W2f, b2f, Wc, bc,
          block):
    n, d = acc_up.shape
    h = W1u.shape[1]
    grid = n // block
    wspec = pl.BlockSpec((d, h), lambda i: (0, 0))
    bspec = pl.BlockSpec((1, h), lambda i: (0, 0))
    nspec = pl.BlockSpec((block, d), lambda i: (i, 0))
    return pl.pallas_call(
        _mlp_body,
        grid=(grid,),
        in_specs=[nspec, nspec,
                  wspec, bspec, wspec, bspec,
                  wspec, bspec, wspec, bspec,
                  wspec, wspec, bspec],
        out_specs=pl.BlockSpec((block, h), lambda i: (i, 0)),
        out_shape=jax.ShapeDtypeStruct((n, h), jnp.float32),
    )(acc_up, acc_f,
      W1u, b1u.reshape(1, -1), W2u, b2u.reshape(1, -1),
      W1f, b1f.reshape(1, -1), W2f, b2f.reshape(1, -1),
      Wc[:d], Wc[d:], bc.reshape(1, -1))


_CH = 80


def _sc_both(face_attr, xw, aw, x, fsrc3, fdst3, usrc3, udst3):
    n, d = x.shape
    ns_chk, nch, ch = usrc3.shape
    e = ns_chk * nch * ch
    info = plsc.get_sparse_core_info()
    ns = info.num_subcores
    ept = e // ns
    half = n // 2
    trash = half
    rpt = (half // ns) // 8 * 8
    tail = half - ns * rpt

    mesh = plsc.VectorSubcoreMesh(core_axis_name="c", subcore_axis_name="s")

    buf_shape = pltpu.VMEM((ch, d), jnp.float32)
    scratch = [
        pltpu.VMEM((nch, ch), jnp.int32),
        pltpu.VMEM((nch, ch), jnp.int32),
        buf_shape,
        buf_shape,
        pltpu.VMEM_SHARED((half + 8, d), jnp.float32),
    ] + [pltpu.SemaphoreType.DMA] * 2

    out_sds = jax.ShapeDtypeStruct((n, d), jnp.float32)

    @functools.partial(
        pl.kernel,
        mesh=mesh,
        out_type=(out_sds, out_sds),
        scratch_types=scratch,
    )
    def k(fa_hbm, xw_hbm, aw_hbm, fsrc_hbm, fdst_hbm, usrc_hbm, udst_hbm,
          x_hbm, out_f, out_up, srcb, dstb, gbuf, abuf, acc, gsem, asem):
        cid = lax.axis_index("c")
        sid = lax.axis_index("s")
        lo = cid * half
        r0 = sid * rpt
        ebase = sid * ept

        for table_hbm, src_hbm, dst_hbm, out, haw in (
                (fa_hbm, fsrc_hbm, fdst_hbm, out_f, False),
                (xw_hbm, usrc_hbm, udst_hbm, out_up, True)):

            with jax.named_scope("ph_init"):
                pltpu.sync_copy(x_hbm.at[pl.ds(lo + r0, rpt)],
                                acc.at[pl.ds(r0, rpt)])

            @pl.when(sid == ns - 1)
            def _init_tail():
                t = ns * rpt
                pltpu.sync_copy(x_hbm.at[pl.ds(lo + t, tail)],
                                acc.at[pl.ds(t, tail)])

            with jax.named_scope("ph_stage"):
                pltpu.sync_copy(src_hbm.at[sid], srcb)
                pltpu.sync_copy(dst_hbm.at[sid], dstb)

            @plsc.parallel_loop(0, nch, 1, unroll=2)
            def _clamp(c):
                for kk in range(ch // 16):
                    s = pl.ds(kk * 16, 16)
                    dv = dstb[c, s]
                    keep = (dv >= lo) & (dv < lo + half)
                    dstb[c, s] = jnp.where(keep, dv - lo, trash)

            plsc.subcore_barrier()

            with jax.named_scope("ph_stream_open"):
                pltpu.trace_value("stream_begin", sid)

            @pl.loop(0, nch)
            def _(c):
                gd = pltpu.async_copy(table_hbm.at[srcb.at[c]], gbuf, gsem)
                if haw:
                    ad = pltpu.async_copy(
                        aw_hbm.at[pl.ds(ebase + c * ch, ch)], abuf, asem)
                gd.wait()
                if haw:
                    ad.wait()

                    @plsc.parallel_loop(0, ch, 1, unroll=2)
                    def _(r):
                        for kk in range(d // 16):
                            s = pl.ds(kk * 16, 16)
                            abuf[r, s] = jnp.maximum(
                                abuf[r, s] + gbuf[r, s], 0.0)

                    pltpu.sync_copy(abuf, acc.at[dstb.at[c]], add=True)
                else:
                    pltpu.sync_copy(gbuf, acc.at[dstb.at[c]], add=True)

            plsc.subcore_barrier()

            with jax.named_scope("ph_flush"):
                pltpu.sync_copy(acc.at[pl.ds(r0, rpt)],
                                out.at[pl.ds(lo + r0, rpt)])

            @pl.when(sid == ns - 1)
            def _flush_tail():
                t = ns * rpt
                pltpu.sync_copy(acc.at[pl.ds(t, tail)],
                                out.at[pl.ds(lo + t, tail)])

            plsc.subcore_barrier()

    return k(face_attr, xw, aw, fsrc3, fdst3, usrc3, udst3, x)


def _idx3(idx, ns, nch, ch):
    return idx[0].reshape(ns, nch, ch), idx[1].reshape(ns, nch, ch)


def kernel(x, up_index, up_attr, face_index, face_attr,
           W_msg, b_msg, W1u, b1u, W2u, b2u, W1f, b1f, W2f, b2f, Wc, bc):
    n, d = x.shape
    e = up_attr.shape[0]

    ns = plsc.get_sparse_core_info().num_subcores
    nch = e // (ns * _CH)
    usrc3, udst3 = _idx3(up_index, ns, nch, _CH)
    fsrc3, fdst3 = _idx3(face_index, ns, nch, _CH)

    xw = _mm(x, W_msg[:d])
    aw = _mm_bias(up_attr, W_msg[d:], b_msg, block=2000)

    acc_f, acc_up = _sc_both(face_attr, xw, aw, x,
                             fsrc3, fdst3, usrc3, udst3)

    return _mlps(acc_up, acc_f, W1u, b1u, W2u, b2u, W1f, b1f, W2f, b2f,
                 Wc, bc, block=1000)


# speedup vs baseline: 1.0799x; 1.0007x over previous
import functools

import jax
import jax.numpy as jnp
from jax import lax
from jax.experimental import pallas as pl
from jax.experimental.pallas import tpu as pltpu
from jax.experimental.pallas import tpu_sc as plsc


def _mm_body(x_ref, w_ref, o_ref):
    o_ref[...] = jnp.dot(x_ref[...], w_ref[...], preferred_element_type=jnp.float32)


def _mm(x, w):
    n, d = x.shape
    return pl.pallas_call(
        _mm_body,
        out_shape=jax.ShapeDtypeStruct((n, w.shape[1]), jnp.float32),
    )(x, w)


def _mm_bias_body(x_ref, w_ref, b_ref, o_ref):
    o_ref[...] = (
        jnp.dot(x_ref[...], w_ref[...], preferred_element_type=jnp.float32)
        + b_ref[...]
    )


def _mm_bias(x, w, b, block):
    n, d = x.shape
    h = w.shape[1]
    grid = n // block
    return pl.pallas_call(
        _mm_bias_body,
        grid=(grid,),
        in_specs=[
            pl.BlockSpec((block, d), lambda i: (i, 0)),
            pl.BlockSpec((d, h), lambda i: (0, 0)),
            pl.BlockSpec((1, h), lambda i: (0, 0)),
        ],
        out_specs=pl.BlockSpec((block, h), lambda i: (i, 0)),
        out_shape=jax.ShapeDtypeStruct((n, h), jnp.float32),
    )(x, w, b.reshape(1, -1))


def _mlp_body(au_ref, af_ref, w1u_ref, b1u_ref, w2u_ref, b2u_ref,
              w1f_ref, b1f_ref, w2f_ref, b2f_ref, wc1_ref, wc2_ref, bc_ref,
              o_ref):
    f32 = jnp.float32
    hu = jnp.maximum(
        jnp.dot(au_ref[...], w1u_ref[...], preferred_element_type=f32)
        + b1u_ref[...], 0.0)
    ou = jnp.maximum(
        jnp.dot(hu, w2u_ref[...], preferred_element_type=f32) + b2u_ref[...],
        0.0)
    hf = jnp.maximum(
        jnp.dot(af_ref[...], w1f_ref[...], preferred_element_type=f32)
        + b1f_ref[...], 0.0)
    of = jnp.maximum(
        jnp.dot(hf, w2f_ref[...], preferred_element_type=f32) + b2f_ref[...],
        0.0)
    o_ref[...] = jnp.maximum(
        jnp.dot(ou, wc1_ref[...], preferred_element_type=f32)
        + jnp.dot(of, wc2_ref[...], preferred_element_type=f32)
        + bc_ref[...], 0.0)


def _mlps(acc_up, acc_f, W1u, b1u, W2u, b2u, W1f, b1f, W2f, b2f, Wc, bc,
          block):
    n, d = acc_up.shape
    h = W1u.shape[1]
    grid = n // block
    wspec = pl.BlockSpec((d, h), lambda i: (0, 0))
    bspec = pl.BlockSpec((1, h), lambda i: (0, 0))
    nspec = pl.BlockSpec((block, d), lambda i: (i, 0))
    return pl.pallas_call(
        _mlp_body,
        grid=(grid,),
        in_specs=[nspec, nspec,
                  wspec, bspec, wspec, bspec,
                  wspec, bspec, wspec, bspec,
                  wspec, wspec, bspec],
        out_specs=pl.BlockSpec((block, h), lambda i: (i, 0)),
        out_shape=jax.ShapeDtypeStruct((n, h), jnp.float32),
    )(acc_up, acc_f,
      W1u, b1u.reshape(1, -1), W2u, b2u.reshape(1, -1),
      W1f, b1f.reshape(1, -1), W2f, b2f.reshape(1, -1),
      Wc[:d], Wc[d:], bc.reshape(1, -1))


_CH = 80


def _sc_both(face_attr, xw, aw, x, fsrc3, fdst3, usrc3, udst3):
    n, d = x.shape
    ns_chk, nch, ch = usrc3.shape
    e = ns_chk * nch * ch
    info = plsc.get_sparse_core_info()
    ns = info.num_subcores
    ept = e // ns
    half = n // 2
    trash = half
    rpt = (half // ns) // 8 * 8
    tail = half - ns * rpt

    mesh = plsc.VectorSubcoreMesh(core_axis_name="c", subcore_axis_name="s")

    buf_shape = pltpu.VMEM((ch, d), jnp.float32)
    scratch = [
        pltpu.VMEM((nch, ch), jnp.int32),
        pltpu.VMEM((nch, ch), jnp.int32),
        buf_shape,
        buf_shape,
        pltpu.VMEM_SHARED((half + 8, d), jnp.float32),
    ] + [pltpu.SemaphoreType.DMA] * 2

    out_sds = jax.ShapeDtypeStruct((n, d), jnp.float32)

    @functools.partial(
        pl.kernel,
        mesh=mesh,
        out_type=(out_sds, out_sds),
        scratch_types=scratch,
    )
    def k(fa_hbm, xw_hbm, aw_hbm, fsrc_hbm, fdst_hbm, usrc_hbm, udst_hbm,
          x_hbm, out_f, out_up, srcb, dstb, gbuf, abuf, acc, gsem, asem):
        cid = lax.axis_index("c")
        sid = lax.axis_index("s")
        lo = cid * half
        r0 = sid * rpt
        ebase = sid * ept

        for table_hbm, src_hbm, dst_hbm, out, haw in (
                (fa_hbm, fsrc_hbm, fdst_hbm, out_f, False),
                (xw_hbm, usrc_hbm, udst_hbm, out_up, True)):

            pltpu.sync_copy(x_hbm.at[pl.ds(lo + r0, rpt)],
                            acc.at[pl.ds(r0, rpt)])

            @pl.when(sid == ns - 1)
            def _init_tail():
                t = ns * rpt
                pltpu.sync_copy(x_hbm.at[pl.ds(lo + t, tail)],
                                acc.at[pl.ds(t, tail)])

            pltpu.sync_copy(src_hbm.at[sid], srcb)
            pltpu.sync_copy(dst_hbm.at[sid], dstb)

            @plsc.parallel_loop(0, nch, 1, unroll=2)
            def _clamp(c):
                for kk in range(ch // 16):
                    s = pl.ds(kk * 16, 16)
                    dv = dstb[c, s]
                    keep = (dv >= lo) & (dv < lo + half)
                    dstb[c, s] = jnp.where(keep, dv - lo, trash)

            plsc.subcore_barrier()

            @pl.loop(0, nch)
            def _(c):
                gd = pltpu.async_copy(table_hbm.at[srcb.at[c]], gbuf, gsem)
                if haw:
                    ad = pltpu.async_copy(
                        aw_hbm.at[pl.ds(ebase + c * ch, ch)], abuf, asem)
                gd.wait()
                if haw:
                    ad.wait()

                    @plsc.parallel_loop(0, ch, 1, unroll=4)
                    def _(r):
                        for kk in range(d // 16):
                            s = pl.ds(kk * 16, 16)
                            abuf[r, s] = jnp.maximum(
                                abuf[r, s] + gbuf[r, s], 0.0)

                    pltpu.sync_copy(abuf, acc.at[dstb.at[c]], add=True)
                else:
                    pltpu.sync_copy(gbuf, acc.at[dstb.at[c]], add=True)

            plsc.subcore_barrier()

            pltpu.sync_copy(acc.at[pl.ds(r0, rpt)],
                            out.at[pl.ds(lo + r0, rpt)])

            @pl.when(sid == ns - 1)
            def _flush_tail():
                t = ns * rpt
                pltpu.sync_copy(acc.at[pl.ds(t, tail)],
                                out.at[pl.ds(lo + t, tail)])

            plsc.subcore_barrier()

    return k(face_attr, xw, aw, fsrc3, fdst3, usrc3, udst3, x)


def _idx3(idx, ns, nch, ch):
    return idx[0].reshape(ns, nch, ch), idx[1].reshape(ns, nch, ch)


def kernel(x, up_index, up_attr, face_index, face_attr,
           W_msg, b_msg, W1u, b1u, W2u, b2u, W1f, b1f, W2f, b2f, Wc, bc):
    n, d = x.shape
    e = up_attr.shape[0]

    ns = plsc.get_sparse_core_info().num_subcores
    nch = e // (ns * _CH)
    usrc3, udst3 = _idx3(up_index, ns, nch, _CH)
    fsrc3, fdst3 = _idx3(face_index, ns, nch, _CH)

    xw = _mm(x, W_msg[:d])
    aw = _mm_bias(up_attr, W_msg[d:], b_msg, block=2000)

    acc_f, acc_up = _sc_both(face_attr, xw, aw, x,
                             fsrc3, fdst3, usrc3, udst3)

    return _mlps(acc_up, acc_f, W1u, b1u, W2u, b2u, W1f, b1f, W2f, b2f,
                 Wc, bc, block=1000)


# speedup vs baseline: 1.0908x; 1.0101x over previous
import functools

import jax
import jax.numpy as jnp
from jax import lax
from jax.experimental import pallas as pl
from jax.experimental.pallas import tpu as pltpu
from jax.experimental.pallas import tpu_sc as plsc


def _mm_body(x_ref, w_ref, o_ref):
    o_ref[...] = jnp.dot(x_ref[...], w_ref[...], preferred_element_type=jnp.float32)


def _mm(x, w):
    n, d = x.shape
    return pl.pallas_call(
        _mm_body,
        out_shape=jax.ShapeDtypeStruct((n, w.shape[1]), jnp.float32),
    )(x, w)


def _mm_bias_body(x_ref, w_ref, b_ref, o_ref):
    o_ref[...] = (
        jnp.dot(x_ref[...], w_ref[...], preferred_element_type=jnp.float32)
        + b_ref[...]
    )


def _mm_bias(x, w, b, block):
    n, d = x.shape
    h = w.shape[1]
    grid = n // block
    return pl.pallas_call(
        _mm_bias_body,
        grid=(grid,),
        in_specs=[
            pl.BlockSpec((block, d), lambda i: (i, 0)),
            pl.BlockSpec((d, h), lambda i: (0, 0)),
            pl.BlockSpec((1, h), lambda i: (0, 0)),
        ],
        out_specs=pl.BlockSpec((block, h), lambda i: (i, 0)),
        out_shape=jax.ShapeDtypeStruct((n, h), jnp.float32),
    )(x, w, b.reshape(1, -1))


def _mlp_body(au_ref, af_ref, w1u_ref, b1u_ref, w2u_ref, b2u_ref,
              w1f_ref, b1f_ref, w2f_ref, b2f_ref, wc1_ref, wc2_ref, bc_ref,
              o_ref):
    f32 = jnp.float32
    hu = jnp.maximum(
        jnp.dot(au_ref[...], w1u_ref[...], preferred_element_type=f32)
        + b1u_ref[...], 0.0)
    ou = jnp.maximum(
        jnp.dot(hu, w2u_ref[...], preferred_element_type=f32) + b2u_ref[...],
        0.0)
    hf = jnp.maximum(
        jnp.dot(af_ref[...], w1f_ref[...], preferred_element_type=f32)
        + b1f_ref[...], 0.0)
    of = jnp.maximum(
        jnp.dot(hf, w2f_ref[...], preferred_element_type=f32) + b2f_ref[...],
        0.0)
    o_ref[...] = jnp.maximum(
        jnp.dot(ou, wc1_ref[...], preferred_element_type=f32)
        + jnp.dot(of, wc2_ref[...], preferred_element_type=f32)
        + bc_ref[...], 0.0)


def _mlps(acc_up, acc_f, W1u, b1u, W2u, b2u, W1f, b1f, W2f, b2f, Wc, bc,
          block):
    n, d = acc_up.shape
    h = W1u.shape[1]
    grid = n // block
    wspec = pl.BlockSpec((d, h), lambda i: (0, 0))
    bspec = pl.BlockSpec((1, h), lambda i: (0, 0))
    nspec = pl.BlockSpec((block, d), lambda i: (i, 0))
    return pl.pallas_call(
        _mlp_body,
        grid=(grid,),
        in_specs=[nspec, nspec,
                  wspec, bspec, wspec, bspec,
                  wspec, bspec, wspec, bspec,
                  wspec, wspec, bspec],
        out_specs=pl.BlockSpec((block, h), lambda i: (i, 0)),
        out_shape=jax.ShapeDtypeStruct((n, h), jnp.float32),
    )(acc_up, acc_f,
      W1u, b1u.reshape(1, -1), W2u, b2u.reshape(1, -1),
      W1f, b1f.reshape(1, -1), W2f, b2f.reshape(1, -1),
      Wc[:d], Wc[d:], bc.reshape(1, -1))


_CH = 80


def _sc_task(table, aw, x, src3, dst3):
    n, d = x.shape
    ns_chk, nch, ch = src3.shape
    e = ns_chk * nch * ch
    info = plsc.get_sparse_core_info()
    ns = info.num_subcores
    ept = e // ns
    half = n // 2
    trash = half
    rpt = (half // ns) // 8 * 8
    tail = half - ns * rpt

    mesh = plsc.VectorSubcoreMesh(core_axis_name="c", subcore_axis_name="s")

    buf_shape = pltpu.VMEM((ch, d), jnp.float32)
    scratch = [
        pltpu.VMEM((nch, ch), jnp.int32),
        pltpu.VMEM((nch, ch), jnp.int32),
        buf_shape,
        buf_shape,
        pltpu.VMEM_SHARED((half + 8, d), jnp.float32),
    ] + [pltpu.SemaphoreType.DMA] * 2

    haw = aw is not None

    @functools.partial(
        pl.kernel,
        mesh=mesh,
        out_type=jax.ShapeDtypeStruct((n, d), jnp.float32),
        scratch_types=scratch,
    )
    def k(table_hbm, aw_hbm, src_hbm, dst_hbm, x_hbm,
          out, srcb, dstb, gbuf, abuf, acc, gsem, asem):
        cid = lax.axis_index("c")
        sid = lax.axis_index("s")
        lo = cid * half
        r0 = sid * rpt
        ebase = sid * ept


        pltpu.sync_copy(x_hbm.at[pl.ds(lo + r0, rpt)],
                        acc.at[pl.ds(r0, rpt)])

        @pl.when(sid == ns - 1)
        def _init_tail():
            t = ns * rpt
            pltpu.sync_copy(x_hbm.at[pl.ds(lo + t, tail)],
                            acc.at[pl.ds(t, tail)])

        pltpu.sync_copy(src_hbm.at[sid], srcb)
        pltpu.sync_copy(dst_hbm.at[sid], dstb)

        @plsc.parallel_loop(0, nch, 1, unroll=2)
        def _clamp(c):
            for kk in range(ch // 16):
                s = pl.ds(kk * 16, 16)
                dv = dstb[c, s]
                keep = (dv >= lo) & (dv < lo + half)
                dstb[c, s] = jnp.where(keep, dv - lo, trash)

        plsc.subcore_barrier()

        @pl.loop(0, nch)
        def _(c):
            gd = pltpu.async_copy(table_hbm.at[srcb.at[c]], gbuf, gsem)
            if haw:
                ad = pltpu.async_copy(
                    aw_hbm.at[pl.ds(ebase + c * ch, ch)], abuf, asem)
            gd.wait()
            if haw:
                ad.wait()

                @plsc.parallel_loop(0, ch, 1, unroll=4)
                def _(r):
                    for kk in range(d // 16):
                        s = pl.ds(kk * 16, 16)
                        abuf[r, s] = jnp.maximum(
                            abuf[r, s] + gbuf[r, s], 0.0)

                pltpu.sync_copy(abuf, acc.at[dstb.at[c]], add=True)
            else:
                pltpu.sync_copy(gbuf, acc.at[dstb.at[c]], add=True)

        plsc.subcore_barrier()

        pltpu.sync_copy(acc.at[pl.ds(r0, rpt)],
                        out.at[pl.ds(lo + r0, rpt)])

        @pl.when(sid == ns - 1)
        def _flush_tail():
            t = ns * rpt
            pltpu.sync_copy(acc.at[pl.ds(t, tail)],
                            out.at[pl.ds(lo + t, tail)])

        plsc.subcore_barrier()

    return k(table, aw if haw else table, src3, dst3, x)


def _idx3(idx, ns, nch, ch):
    return idx[0].reshape(ns, nch, ch), idx[1].reshape(ns, nch, ch)


def kernel(x, up_index, up_attr, face_index, face_attr,
           W_msg, b_msg, W1u, b1u, W2u, b2u, W1f, b1f, W2f, b2f, Wc, bc):
    n, d = x.shape
    e = up_attr.shape[0]

    ns = plsc.get_sparse_core_info().num_subcores
    nch = e // (ns * _CH)
    usrc3, udst3 = _idx3(up_index, ns, nch, _CH)
    fsrc3, fdst3 = _idx3(face_index, ns, nch, _CH)

    xw = _mm(x, W_msg[:d])
    aw = _mm_bias(up_attr, W_msg[d:], b_msg, block=2000)

    acc_f = _sc_task(face_attr, None, x, fsrc3, fdst3)
    acc_up = _sc_task(xw, aw, x, usrc3, udst3)

    return _mlps(acc_up, acc_f, W1u, b1u, W2u, b2u, W1f, b1f, W2f, b2f,
                 Wc, bc, block=1000)
